# Initial kernel scaffold; baseline (speedup 1.0000x reference)
import functools

import jax
import jax.numpy as jnp
from jax import lax
from jax.experimental import pallas as pl
from jax.experimental.pallas import tpu as pltpu
from jax.experimental.pallas import tpu_sc as plsc

N_NODES = 10000
N_PAD = 10240
E_EDGES = 320000
D_IN = 128
D_OUT = 128
N_WORKERS = 32
EDGES_PER_W = E_EDGES // N_WORKERS
CHUNK = 80
N_CHUNKS = EDGES_PER_W // CHUNK
ROWS_PER_TILE = N_PAD // 16


def _mm_body(x_ref, wb_ref, wl_ref, b_ref, yb_ref, yl_ref):
    x = x_ref[...]
    yb_ref[...] = jnp.dot(x, wb_ref[...], preferred_element_type=jnp.float32)
    yl_ref[...] = (
        jnp.dot(x, wl_ref[...], preferred_element_type=jnp.float32) + b_ref[...]
    )


def _final_body(p0_ref, p1_ref, yl_ref, o_ref):
    o_ref[...] = p0_ref[...] + p1_ref[...] + yl_ref[...]


def _sc_body(yb_hbm, src_hbm, dst_hbm, et_hbm, c0t_hbm, c1t_hbm,
             p0_hbm, p1_hbm,
             srcv, dstv, etv, c0v, c1v, rows, mbuf, c0tab, c1tab, acc, sem):
    cid = lax.axis_index("c")
    sid = lax.axis_index("s")
    wid = cid * 16 + sid

    pltpu.sync_copy(src_hbm.at[wid], srcv)
    pltpu.sync_copy(dst_hbm.at[wid], dstv)
    pltpu.sync_copy(et_hbm.at[wid], etv)
    pltpu.sync_copy(c0t_hbm, c0tab)
    pltpu.sync_copy(c1t_hbm, c1tab)

    def _coef(c, carry):
        for g in range(CHUNK // 16):
            et = etv[c, pl.ds(g * 16, 16)]
            c0v[c, pl.ds(g * 16, 16)] = plsc.load_gather(c0tab, [et])
            c1v[c, pl.ds(g * 16, 16)] = plsc.load_gather(c1tab, [et])
        return carry
    lax.fori_loop(0, N_CHUNKS, _coef, 0)

    def _zrow(r, carry):
        for j in range(8):
            mbuf[r, pl.ds(j * 16, 16)] = jnp.zeros((16,), jnp.float32)
        return carry
    lax.fori_loop(0, CHUNK, _zrow, 0)
    for k in range(ROWS_PER_TILE // CHUNK):
        pltpu.sync_copy(mbuf, acc.at[pl.ds(sid * ROWS_PER_TILE + k * CHUNK, CHUNK)])
    plsc.subcore_barrier()

    def _chunk(c, carry):
        pltpu.async_copy(yb_hbm.at[srcv.at[c]], rows, sem).wait()

        def _edge(e, ecarry):
            c0 = c0v[c, e]
            c1 = c1v[c, e]
            for j in range(8):
                y0 = rows[e, pl.ds(j * 16, 16)]
                y1 = rows[e, pl.ds(128 + j * 16, 16)]
                mbuf[e, pl.ds(j * 16, 16)] = c0 * y0 + c1 * y1
            return ecarry
        lax.fori_loop(0, CHUNK, _edge, 0)

        pltpu.sync_copy(mbuf, acc.at[dstv.at[c]], add=True)
        return carry
    lax.fori_loop(0, N_CHUNKS, _chunk, 0)
    plsc.subcore_barrier()

    row0 = sid * ROWS_PER_TILE

    @pl.when(cid == 0)
    def _():
        pltpu.sync_copy(acc.at[pl.ds(row0, ROWS_PER_TILE)],
                        p0_hbm.at[pl.ds(row0, ROWS_PER_TILE)])

    @pl.when(cid == 1)
    def _():
        pltpu.sync_copy(acc.at[pl.ds(row0, ROWS_PER_TILE)],
                        p1_hbm.at[pl.ds(row0, ROWS_PER_TILE)])


_sc_call = pl.kernel(
    _sc_body,
    out_type=[jax.ShapeDtypeStruct((N_PAD, D_OUT), jnp.float32)] * 2,
    mesh=plsc.VectorSubcoreMesh(core_axis_name="c", subcore_axis_name="s"),
    scratch_types=[
        pltpu.VMEM((N_CHUNKS, CHUNK), jnp.int32),
        pltpu.VMEM((N_CHUNKS, CHUNK), jnp.int32),
        pltpu.VMEM((N_CHUNKS, CHUNK), jnp.int32),
        pltpu.VMEM((N_CHUNKS, CHUNK), jnp.float32),
        pltpu.VMEM((N_CHUNKS, CHUNK), jnp.float32),
        pltpu.VMEM((CHUNK, 2 * D_IN), jnp.float32),
        pltpu.VMEM((CHUNK, D_OUT), jnp.float32),
        pltpu.VMEM((16,), jnp.float32),
        pltpu.VMEM((16,), jnp.float32),
        pltpu.VMEM_SHARED((N_PAD, D_OUT), jnp.float32),
        pltpu.SemaphoreType.DMA,
    ],
)


@jax.jit
def kernel(feat, edge_index, etypes, W, coeff, h_bias, loop_weight):
    feat_p = jnp.zeros((N_PAD, D_IN), jnp.float32).at[:N_NODES].set(feat)
    wb = jnp.concatenate([W[0], W[1]], axis=1)
    bias2d = h_bias.reshape(1, D_OUT)

    grid = N_PAD // 512
    yb, yl = pl.pallas_call(
        _mm_body,
        grid=(grid,),
        in_specs=[
            pl.BlockSpec((512, D_IN), lambda i: (i, 0)),
            pl.BlockSpec((D_IN, 2 * D_OUT), lambda i: (0, 0)),
            pl.BlockSpec((D_IN, D_OUT), lambda i: (0, 0)),
            pl.BlockSpec((1, D_OUT), lambda i: (0, 0)),
        ],
        out_specs=[
            pl.BlockSpec((512, 2 * D_OUT), lambda i: (i, 0)),
            pl.BlockSpec((512, D_OUT), lambda i: (i, 0)),
        ],
        out_shape=[
            jax.ShapeDtypeStruct((N_PAD, 2 * D_OUT), jnp.float32),
            jax.ShapeDtypeStruct((N_PAD, D_OUT), jnp.float32),
        ],
    )(feat_p, wb, loop_weight, bias2d)

    src3 = edge_index[0].reshape(N_WORKERS, N_CHUNKS, CHUNK)
    dst3 = edge_index[1].reshape(N_WORKERS, N_CHUNKS, CHUNK)
    et3 = etypes.reshape(N_WORKERS, N_CHUNKS, CHUNK)
    c0t = jnp.zeros((16,), jnp.float32).at[:coeff.shape[0]].set(coeff[:, 0])
    c1t = jnp.zeros((16,), jnp.float32).at[:coeff.shape[0]].set(coeff[:, 1])

    p0, p1 = _sc_call(yb, src3, dst3, et3, c0t, c1t)

    out = pl.pallas_call(
        _final_body,
        grid=(grid,),
        in_specs=[pl.BlockSpec((512, D_OUT), lambda i: (i, 0))] * 3,
        out_specs=pl.BlockSpec((512, D_OUT), lambda i: (i, 0)),
        out_shape=jax.ShapeDtypeStruct((N_PAD, D_OUT), jnp.float32),
    )(p0, p1, yl)
    return out[:N_NODES]


# speedup vs baseline: 3.6324x; 3.6324x over previous
import functools

import jax
import jax.numpy as jnp
from jax import lax
from jax.experimental import pallas as pl
from jax.experimental.pallas import tpu as pltpu
from jax.experimental.pallas import tpu_sc as plsc

N_NODES = 10000
N_PAD = 10240
E_EDGES = 320000
D_IN = 128
D_OUT = 128
N_WORKERS = 32
EDGES_PER_W = E_EDGES // N_WORKERS
CHUNK = 80
N_CHUNKS = EDGES_PER_W // CHUNK
ROWS_PER_TILE = N_PAD // 16


def _mm_body(x_ref, wb_ref, wl_ref, b_ref, yb_ref, yl_ref):
    x = x_ref[...]
    yb_ref[...] = jnp.dot(x, wb_ref[...], preferred_element_type=jnp.float32)
    yl_ref[...] = (
        jnp.dot(x, wl_ref[...], preferred_element_type=jnp.float32) + b_ref[...]
    )


def _final_body(p0_ref, p1_ref, yl_ref, o_ref):
    o_ref[...] = p0_ref[...] + p1_ref[...] + yl_ref[...]


def _sc_body(yb_hbm, src_hbm, dst_hbm, et_hbm, c0t_hbm, c1t_hbm,
             p0_hbm, p1_hbm,
             srcv, dstv, etv, rows, mbuf, c0tab, c1tab, acc, sem):
    cid = lax.axis_index("c")
    sid = lax.axis_index("s")
    wid = cid * 16 + sid

    pltpu.sync_copy(c0t_hbm, c0tab)
    pltpu.sync_copy(c1t_hbm, c1tab)
    c0t_vec = c0tab[...]
    c1t_vec = c1tab[...]
    s0 = [c0t_vec[r] for r in range(8)]
    s1 = [c1t_vec[r] for r in range(8)]

    def _zrow(r, carry):
        for j in range(8):
            mbuf[r, pl.ds(j * 16, 16)] = jnp.zeros((16,), jnp.float32)
        return carry
    lax.fori_loop(0, CHUNK, _zrow, 0)
    for k in range(ROWS_PER_TILE // CHUNK):
        pltpu.sync_copy(mbuf, acc.at[pl.ds(sid * ROWS_PER_TILE + k * CHUNK, CHUNK)])
    plsc.subcore_barrier()

    def _chunk(c, carry):
        pltpu.sync_copy(src_hbm.at[wid, c], srcv)
        pltpu.sync_copy(dst_hbm.at[wid, c], dstv)
        pltpu.sync_copy(et_hbm.at[wid, c], etv)
        pltpu.async_copy(yb_hbm.at[srcv], rows, sem).wait()

        for g in range(CHUNK // 16):
            et = etv[pl.ds(g * 16, 16)]
            c0 = jnp.zeros((16,), jnp.float32)
            c1 = jnp.zeros((16,), jnp.float32)
            for r in range(8):
                m = et == r
                c0 = jnp.where(m, s0[r], c0)
                c1 = jnp.where(m, s1[r], c1)
            for t in range(16):
                e = g * 16 + t
                c0s = c0[t]
                c1s = c1[t]
                for j in range(8):
                    y0 = rows[e, pl.ds(j * 16, 16)]
                    y1 = rows[e, pl.ds(128 + j * 16, 16)]
                    mbuf[e, pl.ds(j * 16, 16)] = c0s * y0 + c1s * y1

        pltpu.sync_copy(mbuf, acc.at[dstv], add=True)
        return carry
    lax.fori_loop(0, N_CHUNKS, _chunk, 0)
    plsc.subcore_barrier()

    row0 = sid * ROWS_PER_TILE

    @pl.when(cid == 0)
    def _():
        pltpu.sync_copy(acc.at[pl.ds(row0, ROWS_PER_TILE)],
                        p0_hbm.at[pl.ds(row0, ROWS_PER_TILE)])

    @pl.when(cid == 1)
    def _():
        pltpu.sync_copy(acc.at[pl.ds(row0, ROWS_PER_TILE)],
                        p1_hbm.at[pl.ds(row0, ROWS_PER_TILE)])


_sc_call = pl.kernel(
    _sc_body,
    out_type=[jax.ShapeDtypeStruct((N_PAD, D_OUT), jnp.float32)] * 2,
    mesh=plsc.VectorSubcoreMesh(core_axis_name="c", subcore_axis_name="s"),
    scratch_types=[
        pltpu.VMEM((CHUNK,), jnp.int32),
        pltpu.VMEM((CHUNK,), jnp.int32),
        pltpu.VMEM((CHUNK,), jnp.int32),
        pltpu.VMEM((CHUNK, 2 * D_IN), jnp.float32),
        pltpu.VMEM((CHUNK, D_OUT), jnp.float32),
        pltpu.VMEM((16,), jnp.float32),
        pltpu.VMEM((16,), jnp.float32),
        pltpu.VMEM_SHARED((N_PAD, D_OUT), jnp.float32),
        pltpu.SemaphoreType.DMA,
    ],
)


@jax.jit
def kernel(feat, edge_index, etypes, W, coeff, h_bias, loop_weight):
    feat_p = jnp.zeros((N_PAD, D_IN), jnp.float32).at[:N_NODES].set(feat)
    wb = jnp.concatenate([W[0], W[1]], axis=1)
    bias2d = h_bias.reshape(1, D_OUT)

    grid = N_PAD // 512
    yb, yl = pl.pallas_call(
        _mm_body,
        grid=(grid,),
        in_specs=[
            pl.BlockSpec((512, D_IN), lambda i: (i, 0)),
            pl.BlockSpec((D_IN, 2 * D_OUT), lambda i: (0, 0)),
            pl.BlockSpec((D_IN, D_OUT), lambda i: (0, 0)),
            pl.BlockSpec((1, D_OUT), lambda i: (0, 0)),
        ],
        out_specs=[
            pl.BlockSpec((512, 2 * D_OUT), lambda i: (i, 0)),
            pl.BlockSpec((512, D_OUT), lambda i: (i, 0)),
        ],
        out_shape=[
            jax.ShapeDtypeStruct((N_PAD, 2 * D_OUT), jnp.float32),
            jax.ShapeDtypeStruct((N_PAD, D_OUT), jnp.float32),
        ],
    )(feat_p, wb, loop_weight, bias2d)

    src3 = edge_index[0].reshape(N_WORKERS, N_CHUNKS, CHUNK)
    dst3 = edge_index[1].reshape(N_WORKERS, N_CHUNKS, CHUNK)
    et3 = etypes.reshape(N_WORKERS, N_CHUNKS, CHUNK)
    c0t = jnp.zeros((16,), jnp.float32).at[:coeff.shape[0]].set(coeff[:, 0])
    c1t = jnp.zeros((16,), jnp.float32).at[:coeff.shape[0]].set(coeff[:, 1])

    p0, p1 = _sc_call(yb, src3, dst3, et3, c0t, c1t)

    out = pl.pallas_call(
        _final_body,
        grid=(grid,),
        in_specs=[pl.BlockSpec((512, D_OUT), lambda i: (i, 0))] * 3,
        out_specs=pl.BlockSpec((512, D_OUT), lambda i: (i, 0)),
        out_shape=jax.ShapeDtypeStruct((N_PAD, D_OUT), jnp.float32),
    )(p0, p1, yl)
    return out[:N_NODES]
